# baseline (speedup 1.0000x reference)
import jax
import jax.numpy as jnp
from jax.experimental import pallas as pl
from jax.experimental.pallas import tpu as pltpu

_N = 10000
_IN_F = 128
_OUT_F = 128
_ALPHA = 0.5
_BM = 200


def _gcn_block(inp_ref, w_ref, adj_ref, init_ref, out_ref, support_ref):
    i = pl.program_id(0)

    @pl.when(i == 0)
    def _compute_support():
        support_ref[...] = jnp.dot(
            inp_ref[...], w_ref[...], preferred_element_type=jnp.float32
        )

    @pl.when(i > 0)
    def _spmm_block():
        acc = jnp.dot(
            adj_ref[...],
            support_ref[...],
            precision=jax.lax.Precision.DEFAULT,
            preferred_element_type=jnp.float32,
        )
        out_ref[...] = jnp.maximum(acc * _ALPHA + init_ref[...] * (1.0 - _ALPHA), 0.0)


def _lag(i):
    j = jnp.maximum(i - 1, 0)
    return (j, 0)


def kernel(input, adj, init_input, W):
    return pl.pallas_call(
        _gcn_block,
        grid=(_N // _BM + 1,),
        in_specs=[
            pl.BlockSpec((_N, _IN_F), lambda i: (0, 0)),
            pl.BlockSpec((_IN_F, _OUT_F), lambda i: (0, 0)),
            pl.BlockSpec((_BM, _N), _lag),
            pl.BlockSpec((_BM, _OUT_F), _lag),
        ],
        out_specs=pl.BlockSpec((_BM, _OUT_F), _lag),
        out_shape=jax.ShapeDtypeStruct((_N, _OUT_F), jnp.float32),
        scratch_shapes=[pltpu.VMEM((_N, _OUT_F), jnp.float32)],
    )(input, W, adj, init_input)
